# baseline (speedup 1.0000x reference)
import functools

import jax
import jax.numpy as jnp
from jax import lax
from jax.experimental import pallas as pl
from jax.experimental.pallas import tpu as pltpu
from jax.experimental.pallas import tpu_sc as plsc

NUM_KEYS = 1000000
D = 64
TPB = 256


@functools.cache
def _build(Bt, H):
    info = plsc.get_sparse_core_info()
    NC, NS, L = info.num_cores, info.num_subcores, info.num_lanes
    NW = NC * NS
    n_bblk = Bt // TPB
    n_tasks = H * n_bblk
    tasks_pw = n_tasks // NW
    assert Bt % TPB == 0 and n_tasks % NW == 0 and tasks_pw % 2 == 0
    mesh = plsc.VectorSubcoreMesh(core_axis_name="c", subcore_axis_name="s")

    @functools.partial(
        pl.kernel,
        mesh=mesh,
        compiler_params=pltpu.CompilerParams(
            use_tc_tiling_on_sc=True, needs_layout_passes=False
        ),
        out_type=jax.ShapeDtypeStruct((H, D, Bt), jnp.float32),
        scratch_types=[
            pltpu.VMEM((TPB,), jnp.int32),
            pltpu.VMEM((TPB,), jnp.int32),
            pltpu.VMEM((TPB,), jnp.int32),
            pltpu.VMEM((TPB,), jnp.int32),
            pltpu.VMEM((TPB,), jnp.int32),
            pltpu.VMEM((TPB,), jnp.int32),
            pltpu.VMEM((TPB, 128), jnp.float32),
            pltpu.VMEM((TPB, 128), jnp.float32),
            pltpu.VMEM((D, TPB), jnp.float32),
            pltpu.VMEM((D, TPB), jnp.float32),
            pltpu.SemaphoreType.DMA,
            pltpu.SemaphoreType.DMA,
            pltpu.SemaphoreType.DMA,
            pltpu.SemaphoreType.DMA,
            pltpu.SemaphoreType.DMA,
            pltpu.SemaphoreType.DMA,
        ],
    )
    def k(kct_hbm, tab_hbm, out_hbm, ri0, ri1, ib0, ib1, pb0, pb1, gb0, gb1,
          ob0, ob1, g0, g1, s0, s1, i0, i1):
        wid = lax.axis_index("s") * NC + lax.axis_index("c")
        tbase = wid * tasks_pw
        ris, ibs, pbs = (ri0, ri1), (ib0, ib1), (pb0, pb1)
        gbs, obs = (gb0, gb1), (ob0, ob1)
        gsems, ssems, isems = (g0, g1), (s0, s1), (i0, i1)

        def task_hb(t):
            gt = tbase + t
            h = gt // n_bblk
            b0 = (gt % n_bblk) * TPB
            return h, b0

        def fetch_idx(t, z):
            h, b0 = task_hb(t)
            pltpu.async_copy(kct_hbm.at[h, pl.ds(b0, TPB)], ris[z], isems[z])

        def prep(t, z):
            h, b0 = task_hb(t)
            ri, ib, pb = ris[z], ibs[z], pbs[z]
            pltpu.make_async_copy(
                kct_hbm.at[h, pl.ds(b0, TPB)], ri, isems[z]
            ).wait()

            def cbody(i, c):
                v = jnp.clip(ri[pl.ds(i * L, L)], 0, NUM_KEYS - 1)
                ib[pl.ds(i * L, L)] = lax.shift_right_logical(v, 1)
                pb[pl.ds(i * L, L)] = lax.shift_left(v & 1, 6)
                return c

            lax.fori_loop(0, TPB // L, cbody, 0, unroll=4)
            pltpu.async_copy(tab_hbm.at[ib], gbs[z], gsems[z])

        def wait_gather(z):
            pltpu.make_async_copy(tab_hbm.at[ibs[z]], gbs[z], gsems[z]).wait()

        def start_scatter(t, z):
            h, b0 = task_hb(t)
            pltpu.async_copy(obs[z], out_hbm.at[h, :, pl.ds(b0, TPB)], ssems[z])

        def wait_scatter(t, z):
            h, b0 = task_hb(t)
            pltpu.make_async_copy(
                obs[z], out_hbm.at[h, :, pl.ds(b0, TPB)], ssems[z]
            ).wait()

        def compute(z):
            gb, pb, ob = gbs[z], pbs[z], obs[z]
            lanes = lax.iota(jnp.int32, L)
            rots = [(lanes + c) & (L - 1) for c in range(L)]

            def loads(j, t):
                rowsv = lanes + j * L
                parv = pb[pl.ds(j * L, L)]
                colbase = parv + t * L
                return [
                    plsc.load_gather(gb, [rowsv, colbase + rots[c]])
                    for c in range(L)
                ]

            def tanh_b(xs):
                es = [jnp.exp(x + x) for x in xs]
                return [1.0 - 2.0 / (1.0 + e) for e in es]

            def stores(j, t, ys):
                rowsv = lanes + j * L
                for c in range(L):
                    plsc.store_scatter(ob, [rots[c] + t * L, rowsv], ys[c])

            NG = D // L

            def jbody(j, c0):
                for t in range(NG):
                    xs = loads(j, t)
                    ys = tanh_b(xs)
                    stores(j, t, ys)
                return c0

            lax.fori_loop(0, TPB // L, jbody, 0, unroll=2)

        fetch_idx(0, 0)
        fetch_idx(1, 1)
        prep(0, 0)
        prep(1, 1)

        def pipe(h2, c):
            for z in (0, 1):
                t = 2 * h2 + z

                @pl.when(t >= 2)
                def _():
                    wait_scatter(t - 2, z)

                @pl.when(t + 2 < tasks_pw)
                def _():
                    fetch_idx(t + 2, z)

                wait_gather(z)
                compute(z)
                start_scatter(t, z)

                @pl.when(t + 2 < tasks_pw)
                def _():
                    prep(t + 2, z)

            return c

        lax.fori_loop(0, tasks_pw // 2, pipe, 0)

        wait_scatter(tasks_pw - 2, 0)
        wait_scatter(tasks_pw - 1, 1)

    return k


@jax.jit
def kernel(key_codes, weight):
    Bt, H = key_codes.shape
    kct = key_codes.T
    tab = weight.reshape(NUM_KEYS // 2, 2 * D)
    out3 = _build(Bt, H)(kct, tab)
    return out3.transpose(2, 0, 1)


# speedup vs baseline: 1.0089x; 1.0089x over previous
import functools

import jax
import jax.numpy as jnp
from jax import lax
from jax.experimental import pallas as pl
from jax.experimental.pallas import tpu as pltpu
from jax.experimental.pallas import tpu_sc as plsc

NUM_KEYS = 1000000
D = 64
TPB = 256


@functools.cache
def _build(Bt, H):
    info = plsc.get_sparse_core_info()
    NC, NS, L = info.num_cores, info.num_subcores, info.num_lanes
    NW = NC * NS
    n_bblk = Bt // TPB
    n_tasks = H * n_bblk
    tasks_pw = n_tasks // NW
    assert Bt % TPB == 0 and n_tasks % NW == 0 and tasks_pw % 2 == 0
    mesh = plsc.VectorSubcoreMesh(core_axis_name="c", subcore_axis_name="s")

    @functools.partial(
        pl.kernel,
        mesh=mesh,
        compiler_params=pltpu.CompilerParams(
            use_tc_tiling_on_sc=True, needs_layout_passes=False
        ),
        out_type=jax.ShapeDtypeStruct((H, D, Bt), jnp.float32),
        scratch_types=[
            pltpu.VMEM((TPB,), jnp.int32),
            pltpu.VMEM((TPB,), jnp.int32),
            pltpu.VMEM((TPB,), jnp.int32),
            pltpu.VMEM((TPB,), jnp.int32),
            pltpu.VMEM((TPB,), jnp.int32),
            pltpu.VMEM((TPB,), jnp.int32),
            pltpu.VMEM((TPB, 128), jnp.float32),
            pltpu.VMEM((TPB, 128), jnp.float32),
            pltpu.VMEM((D, TPB), jnp.float32),
            pltpu.VMEM((D, TPB), jnp.float32),
            pltpu.SemaphoreType.DMA,
            pltpu.SemaphoreType.DMA,
            pltpu.SemaphoreType.DMA,
            pltpu.SemaphoreType.DMA,
            pltpu.SemaphoreType.DMA,
            pltpu.SemaphoreType.DMA,
        ],
    )
    def k(kct_hbm, tab_hbm, out_hbm, ri0, ri1, ib0, ib1, pb0, pb1, gb0, gb1,
          ob0, ob1, g0, g1, s0, s1, i0, i1):
        wid = lax.axis_index("s") * NC + lax.axis_index("c")
        tbase = wid * tasks_pw
        ris, ibs, pbs = (ri0, ri1), (ib0, ib1), (pb0, pb1)
        gbs, obs = (gb0, gb1), (ob0, ob1)
        gsems, ssems, isems = (g0, g1), (s0, s1), (i0, i1)

        def task_hb(t):
            gt = tbase + t
            h = gt // n_bblk
            b0 = (gt % n_bblk) * TPB
            return h, b0

        def fetch_idx(t, z):
            h, b0 = task_hb(t)
            pltpu.async_copy(kct_hbm.at[h, pl.ds(b0, TPB)], ris[z], isems[z])

        def prep(t, z):
            h, b0 = task_hb(t)
            ri, ib, pb = ris[z], ibs[z], pbs[z]
            pltpu.make_async_copy(
                kct_hbm.at[h, pl.ds(b0, TPB)], ri, isems[z]
            ).wait()

            def cbody(i, c):
                v = jnp.clip(ri[pl.ds(i * L, L)], 0, NUM_KEYS - 1)
                ib[pl.ds(i * L, L)] = lax.shift_right_logical(v, 1)
                pb[pl.ds(i * L, L)] = lax.shift_left(v & 1, 6)
                return c

            lax.fori_loop(0, TPB // L, cbody, 0, unroll=4)
            pltpu.async_copy(tab_hbm.at[ib], gbs[z], gsems[z])

        def wait_gather(z):
            pltpu.make_async_copy(tab_hbm.at[ibs[z]], gbs[z], gsems[z]).wait()

        def start_scatter(t, z):
            h, b0 = task_hb(t)
            pltpu.async_copy(obs[z], out_hbm.at[h, :, pl.ds(b0, TPB)], ssems[z])

        def wait_scatter(t, z):
            h, b0 = task_hb(t)
            pltpu.make_async_copy(
                obs[z], out_hbm.at[h, :, pl.ds(b0, TPB)], ssems[z]
            ).wait()

        def compute(z):
            gb, pb, ob = gbs[z], pbs[z], obs[z]
            lanes = lax.iota(jnp.int32, L)
            rots = [(lanes + c) & (L - 1) for c in range(L)]

            def loads(j, t):
                rowsv = lanes + j * L
                parv = pb[pl.ds(j * L, L)]
                colbase = parv + t * L
                return [
                    plsc.load_gather(gb, [rowsv, colbase + rots[c]])
                    for c in range(L)
                ]

            def tanh_b(xs):
                es = [jnp.exp(x + x) for x in xs]
                return [1.0 - 2.0 / (1.0 + e) for e in es]

            def stores(j, t, ys):
                rowsv = lanes + j * L
                for c in range(L):
                    plsc.store_scatter(ob, [rots[c] + t * L, rowsv], ys[c])

            NG = D // L

            def jbody(j, c0):
                for t in range(NG):
                    xs = loads(j, t)
                    ys = tanh_b(xs)
                    stores(j, t, ys)
                return c0

            lax.fori_loop(0, TPB // L, jbody, 0, unroll=2)

        fetch_idx(0, 0)
        fetch_idx(1, 1)
        prep(0, 0)
        prep(1, 1)
        for z in (0, 1):
            fetch_idx(z + 2, z)
            wait_gather(z)
            compute(z)
            start_scatter(z, z)
            prep(z + 2, z)

        def pipe(h2, c):
            for z in (0, 1):
                t = 2 * h2 + z
                wait_scatter(t - 2, z)

                @pl.when(t + 2 < tasks_pw)
                def _():
                    fetch_idx(t + 2, z)

                wait_gather(z)
                compute(z)
                start_scatter(t, z)

                @pl.when(t + 2 < tasks_pw)
                def _():
                    prep(t + 2, z)

            return c

        lax.fori_loop(1, tasks_pw // 2, pipe, 0)

        wait_scatter(tasks_pw - 2, 0)
        wait_scatter(tasks_pw - 1, 1)

    return k


@jax.jit
def kernel(key_codes, weight):
    Bt, H = key_codes.shape
    kct = key_codes.T
    tab = weight.reshape(NUM_KEYS // 2, 2 * D)
    out3 = _build(Bt, H)(kct, tab)
    return out3.transpose(2, 0, 1)
